# baseline (device time: 56854 ns/iter reference)
import jax
import jax.numpy as jnp
from jax import lax
from jax.experimental import pallas as pl
from jax.experimental.pallas import tpu as pltpu

N_DEV = 8


def kernel(x, Win0, Wout0, Win1, Wout1, Win2, Wout2):
    b_blk, d = x.shape

    def body(x_ref, win0_ref, wout0_ref, win1_ref, wout1_ref, win2_ref,
             wout2_ref, out_ref, xbuf, psend, rsbuf,
             ag_send, ag_recv, rs_send, rs_recv):
        me = lax.axis_index("i")

        bar = pltpu.get_barrier_semaphore()
        for off in range(1, N_DEV):
            t = (me + off) % N_DEV
            pl.semaphore_signal(bar, inc=1, device_id=(t,),
                                device_id_type=pl.DeviceIdType.MESH)
        pl.semaphore_wait(bar, N_DEV - 1)

        xbuf[me] = x_ref[:, :].astype(jnp.bfloat16)

        wins = [win0_ref, win1_ref, win2_ref]
        wouts = [wout0_ref, wout1_ref, wout2_ref]

        def ag_push(target):
            return pltpu.make_async_remote_copy(
                src_ref=xbuf.at[me], dst_ref=xbuf.at[me],
                send_sem=ag_send.at[target], recv_sem=ag_recv.at[me],
                device_id=(target,), device_id_type=pl.DeviceIdType.MESH)

        def ag_recv_desc(sender):
            return pltpu.make_async_remote_copy(
                src_ref=xbuf.at[sender], dst_ref=xbuf.at[sender],
                send_sem=ag_send.at[sender], recv_sem=ag_recv.at[sender],
                device_id=(sender,), device_id_type=pl.DeviceIdType.MESH)

        acc = None
        for l in range(3):
            ag_rdmas = []
            for off in range(1, N_DEV):
                r = ag_push((me + off) % N_DEV)
                r.start()
                ag_rdmas.append(r)

            w_in = wins[l][:, :].astype(jnp.bfloat16)
            w_out = wouts[l][:, :].astype(jnp.bfloat16)

            def block_partial(xb):
                h = jnp.dot(xb, w_in, preferred_element_type=jnp.float32)
                h = jnp.maximum(h, 0.0).astype(jnp.bfloat16)
                return jnp.dot(h, w_out, preferred_element_type=jnp.float32)

            acc = block_partial(xbuf[me])

            rs_rdmas = []
            for off in range(1, N_DEV):
                s = (me + off) % N_DEV
                ag_recv_desc(s).wait_recv()
                psend[s] = block_partial(xbuf[s]).astype(jnp.bfloat16)
                r = pltpu.make_async_remote_copy(
                    src_ref=psend.at[s], dst_ref=rsbuf.at[me],
                    send_sem=rs_send.at[s], recv_sem=rs_recv.at[me],
                    device_id=(s,), device_id_type=pl.DeviceIdType.MESH)
                r.start()
                rs_rdmas.append(r)

            for off in range(1, N_DEV):
                s = (me + off) % N_DEV
                recv = pltpu.make_async_remote_copy(
                    src_ref=rsbuf.at[s], dst_ref=rsbuf.at[s],
                    send_sem=rs_send.at[s], recv_sem=rs_recv.at[s],
                    device_id=(s,), device_id_type=pl.DeviceIdType.MESH)
                recv.wait_recv()
                acc = acc + rsbuf[s].astype(jnp.float32)

            for r in ag_rdmas + rs_rdmas:
                r.wait_send()

            xbuf[me] = acc.astype(jnp.bfloat16)

        fin_rdmas = []
        for off in range(1, N_DEV):
            r = ag_push((me + off) % N_DEV)
            r.start()
            fin_rdmas.append(r)
        out_ref[pl.ds(me * b_blk, b_blk), :] = acc
        for off in range(1, N_DEV):
            s = (me + off) % N_DEV
            ag_recv_desc(s).wait_recv()
            out_ref[pl.ds(s * b_blk, b_blk), :] = xbuf[s].astype(jnp.float32)
        for r in fin_rdmas:
            r.wait_send()

    return pl.pallas_call(
        body,
        out_shape=jax.ShapeDtypeStruct((N_DEV * b_blk, d), jnp.float32),
        in_specs=[pl.BlockSpec(memory_space=pltpu.VMEM)] * 7,
        out_specs=pl.BlockSpec(memory_space=pltpu.VMEM),
        scratch_shapes=[
            pltpu.VMEM((N_DEV, b_blk, d), jnp.bfloat16),
            pltpu.VMEM((N_DEV, b_blk, d), jnp.bfloat16),
            pltpu.VMEM((N_DEV, b_blk, d), jnp.bfloat16),
            pltpu.SemaphoreType.DMA((N_DEV,)),
            pltpu.SemaphoreType.DMA((N_DEV,)),
            pltpu.SemaphoreType.DMA((N_DEV,)),
            pltpu.SemaphoreType.DMA((N_DEV,)),
        ],
        compiler_params=pltpu.CompilerParams(collective_id=0),
    )(x, Win0, Wout0, Win1, Wout1, Win2, Wout2)


# device time: 56697 ns/iter; 1.0028x vs baseline; 1.0028x over previous
import jax
import jax.numpy as jnp
from jax import lax
from jax.experimental import pallas as pl
from jax.experimental.pallas import tpu as pltpu

N_DEV = 8


def kernel(x, Win0, Wout0, Win1, Wout1, Win2, Wout2):
    b_blk, d = x.shape

    def body(x_ref, win0_ref, wout0_ref, win1_ref, wout1_ref, win2_ref,
             wout2_ref, out_ref, xbuf, psend, rsbuf,
             ag_send, ag_recv, rs_send, rs_recv):
        me = lax.axis_index("i")

        bar = pltpu.get_barrier_semaphore()
        for off in range(1, N_DEV):
            t = (me + off) % N_DEV
            pl.semaphore_signal(bar, inc=1, device_id=(t,),
                                device_id_type=pl.DeviceIdType.MESH)
        pl.semaphore_wait(bar, N_DEV - 1)

        xbuf[me] = x_ref[:, :].astype(jnp.bfloat16)

        wins = [win0_ref, win1_ref, win2_ref]
        wouts = [wout0_ref, wout1_ref, wout2_ref]

        def ag_push(target):
            return pltpu.make_async_remote_copy(
                src_ref=xbuf.at[me], dst_ref=xbuf.at[me],
                send_sem=ag_send.at[target], recv_sem=ag_recv.at[me],
                device_id=(target,), device_id_type=pl.DeviceIdType.MESH)

        def ag_recv_desc(sender):
            return pltpu.make_async_remote_copy(
                src_ref=xbuf.at[sender], dst_ref=xbuf.at[sender],
                send_sem=ag_send.at[sender], recv_sem=ag_recv.at[sender],
                device_id=(sender,), device_id_type=pl.DeviceIdType.MESH)

        acc = None
        for l in range(3):
            ag_rdmas = []
            for off in range(1, N_DEV):
                r = ag_push((me + off) % N_DEV)
                r.start()
                ag_rdmas.append(r)

            w_in = wins[l][:, :].astype(jnp.bfloat16)
            w_out = wouts[l][:, :].astype(jnp.bfloat16)

            for off in range(1, N_DEV):
                ag_recv_desc((me + off) % N_DEV).wait_recv()
            x_all = xbuf[:, :, :].reshape(N_DEV * b_blk, d)
            h = jnp.dot(x_all, w_in, preferred_element_type=jnp.float32)
            h = jnp.maximum(h, 0.0).astype(jnp.bfloat16)
            p = jnp.dot(h, w_out, preferred_element_type=jnp.float32)
            psend[:, :, :] = p.astype(jnp.bfloat16).reshape(N_DEV, b_blk, d)

            rs_rdmas = []
            for off in range(1, N_DEV):
                s = (me + off) % N_DEV
                r = pltpu.make_async_remote_copy(
                    src_ref=psend.at[s], dst_ref=rsbuf.at[me],
                    send_sem=rs_send.at[s], recv_sem=rs_recv.at[me],
                    device_id=(s,), device_id_type=pl.DeviceIdType.MESH)
                r.start()
                rs_rdmas.append(r)

            acc = psend[me].astype(jnp.float32)
            for off in range(1, N_DEV):
                s = (me + off) % N_DEV
                recv = pltpu.make_async_remote_copy(
                    src_ref=rsbuf.at[s], dst_ref=rsbuf.at[s],
                    send_sem=rs_send.at[s], recv_sem=rs_recv.at[s],
                    device_id=(s,), device_id_type=pl.DeviceIdType.MESH)
                recv.wait_recv()
                acc = acc + rsbuf[s].astype(jnp.float32)

            for r in ag_rdmas + rs_rdmas:
                r.wait_send()

            xbuf[me] = acc.astype(jnp.bfloat16)

        fin_rdmas = []
        for off in range(1, N_DEV):
            r = ag_push((me + off) % N_DEV)
            r.start()
            fin_rdmas.append(r)
        out_ref[pl.ds(me * b_blk, b_blk), :] = acc
        for off in range(1, N_DEV):
            s = (me + off) % N_DEV
            ag_recv_desc(s).wait_recv()
            out_ref[pl.ds(s * b_blk, b_blk), :] = xbuf[s].astype(jnp.float32)
        for r in fin_rdmas:
            r.wait_send()

    return pl.pallas_call(
        body,
        out_shape=jax.ShapeDtypeStruct((N_DEV * b_blk, d), jnp.float32),
        in_specs=[pl.BlockSpec(memory_space=pltpu.VMEM)] * 7,
        out_specs=pl.BlockSpec(memory_space=pltpu.VMEM),
        scratch_shapes=[
            pltpu.VMEM((N_DEV, b_blk, d), jnp.bfloat16),
            pltpu.VMEM((N_DEV, b_blk, d), jnp.bfloat16),
            pltpu.VMEM((N_DEV, b_blk, d), jnp.bfloat16),
            pltpu.SemaphoreType.DMA((N_DEV,)),
            pltpu.SemaphoreType.DMA((N_DEV,)),
            pltpu.SemaphoreType.DMA((N_DEV,)),
            pltpu.SemaphoreType.DMA((N_DEV,)),
        ],
        compiler_params=pltpu.CompilerParams(collective_id=0),
    )(x, Win0, Wout0, Win1, Wout1, Win2, Wout2)


# device time: 56325 ns/iter; 1.0094x vs baseline; 1.0066x over previous
import jax
import jax.numpy as jnp
from jax import lax
from jax.experimental import pallas as pl
from jax.experimental.pallas import tpu as pltpu

N_DEV = 8


def kernel(x, Win0, Wout0, Win1, Wout1, Win2, Wout2):
    b_blk, d = x.shape
    h_blk = Win0.shape[1]

    def body(x_ref, win0_ref, wout0_ref, win1_ref, wout1_ref, win2_ref,
             wout2_ref, out_ref, xbuf, psend, rsbuf, wibuf, wobuf,
             ag_send, ag_recv, rs_send, rs_recv, wcopy_sems):
        me = lax.axis_index("i")

        wcopies = []
        for i, ref in enumerate([win0_ref, win1_ref, win2_ref]):
            c = pltpu.make_async_copy(ref, wibuf.at[i], wcopy_sems.at[i])
            c.start()
            wcopies.append(c)
        for i, ref in enumerate([wout0_ref, wout1_ref, wout2_ref]):
            c = pltpu.make_async_copy(ref, wobuf.at[i], wcopy_sems.at[3 + i])
            c.start()
            wcopies.append(c)

        bar = pltpu.get_barrier_semaphore()
        for off in range(1, N_DEV):
            t = (me + off) % N_DEV
            pl.semaphore_signal(bar, inc=1, device_id=(t,),
                                device_id_type=pl.DeviceIdType.MESH)
        pl.semaphore_wait(bar, N_DEV - 1)

        xbuf[me] = x_ref[:, :].astype(jnp.bfloat16)

        def ag_push(target):
            return pltpu.make_async_remote_copy(
                src_ref=xbuf.at[me], dst_ref=xbuf.at[me],
                send_sem=ag_send.at[target], recv_sem=ag_recv.at[me],
                device_id=(target,), device_id_type=pl.DeviceIdType.MESH)

        def ag_recv_desc(sender):
            return pltpu.make_async_remote_copy(
                src_ref=xbuf.at[sender], dst_ref=xbuf.at[sender],
                send_sem=ag_send.at[sender], recv_sem=ag_recv.at[sender],
                device_id=(sender,), device_id_type=pl.DeviceIdType.MESH)

        acc = None
        for l in range(3):
            ag_rdmas = []
            for off in range(1, N_DEV):
                r = ag_push((me + off) % N_DEV)
                r.start()
                ag_rdmas.append(r)

            wcopies[l].wait()
            wcopies[3 + l].wait()
            w_in = wibuf[l].astype(jnp.bfloat16)
            w_out = wobuf[l].astype(jnp.bfloat16)

            def chunk_partial(offs):
                xc = jnp.concatenate(
                    [xbuf[(me + o) % N_DEV] for o in offs], axis=0)
                h = jnp.dot(xc, w_in, preferred_element_type=jnp.float32)
                h = jnp.maximum(h, 0.0).astype(jnp.bfloat16)
                return jnp.dot(h, w_out, preferred_element_type=jnp.float32)

            rs_rdmas = []
            for off in (1, 2, 3):
                ag_recv_desc((me + off) % N_DEV).wait_recv()
            p1 = chunk_partial((0, 1, 2, 3))
            for i, off in enumerate((0, 1, 2, 3)):
                s = (me + off) % N_DEV
                psend[s] = p1[i * b_blk:(i + 1) * b_blk].astype(jnp.bfloat16)
                if off == 0:
                    continue
                r = pltpu.make_async_remote_copy(
                    src_ref=psend.at[s], dst_ref=rsbuf.at[me],
                    send_sem=rs_send.at[s], recv_sem=rs_recv.at[me],
                    device_id=(s,), device_id_type=pl.DeviceIdType.MESH)
                r.start()
                rs_rdmas.append(r)

            for off in (4, 5, 6, 7):
                ag_recv_desc((me + off) % N_DEV).wait_recv()
            p2 = chunk_partial((4, 5, 6, 7))
            for i, off in enumerate((4, 5, 6, 7)):
                s = (me + off) % N_DEV
                psend[s] = p2[i * b_blk:(i + 1) * b_blk].astype(jnp.bfloat16)
                r = pltpu.make_async_remote_copy(
                    src_ref=psend.at[s], dst_ref=rsbuf.at[me],
                    send_sem=rs_send.at[s], recv_sem=rs_recv.at[me],
                    device_id=(s,), device_id_type=pl.DeviceIdType.MESH)
                r.start()
                rs_rdmas.append(r)

            acc = p1[0:b_blk]
            for off in range(1, N_DEV):
                s = (me + off) % N_DEV
                recv = pltpu.make_async_remote_copy(
                    src_ref=rsbuf.at[s], dst_ref=rsbuf.at[s],
                    send_sem=rs_send.at[s], recv_sem=rs_recv.at[s],
                    device_id=(s,), device_id_type=pl.DeviceIdType.MESH)
                recv.wait_recv()
                acc = acc + rsbuf[s].astype(jnp.float32)

            for r in ag_rdmas + rs_rdmas:
                r.wait_send()

            xbuf[me] = acc.astype(jnp.bfloat16)

        fin_rdmas = []
        for off in range(1, N_DEV):
            r = ag_push((me + off) % N_DEV)
            r.start()
            fin_rdmas.append(r)
        out_ref[pl.ds(me * b_blk, b_blk), :] = acc
        for off in range(1, N_DEV):
            s = (me + off) % N_DEV
            ag_recv_desc(s).wait_recv()
            out_ref[pl.ds(s * b_blk, b_blk), :] = xbuf[s].astype(jnp.float32)
        for r in fin_rdmas:
            r.wait_send()

    return pl.pallas_call(
        body,
        out_shape=jax.ShapeDtypeStruct((N_DEV * b_blk, d), jnp.float32),
        in_specs=[pl.BlockSpec(memory_space=pltpu.VMEM)]
        + [pl.BlockSpec(memory_space=pltpu.MemorySpace.HBM)] * 6,
        out_specs=pl.BlockSpec(memory_space=pltpu.VMEM),
        scratch_shapes=[
            pltpu.VMEM((N_DEV, b_blk, d), jnp.bfloat16),
            pltpu.VMEM((N_DEV, b_blk, d), jnp.bfloat16),
            pltpu.VMEM((N_DEV, b_blk, d), jnp.bfloat16),
            pltpu.VMEM((3, d, h_blk), jnp.float32),
            pltpu.VMEM((3, h_blk, d), jnp.float32),
            pltpu.SemaphoreType.DMA((N_DEV,)),
            pltpu.SemaphoreType.DMA((N_DEV,)),
            pltpu.SemaphoreType.DMA((N_DEV,)),
            pltpu.SemaphoreType.DMA((N_DEV,)),
            pltpu.SemaphoreType.DMA((6,)),
        ],
        compiler_params=pltpu.CompilerParams(collective_id=0),
    )(x, Win0, Wout0, Win1, Wout1, Win2, Wout2)


# device time: 55292 ns/iter; 1.0283x vs baseline; 1.0187x over previous
import jax
import jax.numpy as jnp
from jax import lax
from jax.experimental import pallas as pl
from jax.experimental.pallas import tpu as pltpu

N_DEV = 8


def kernel(x, Win0, Wout0, Win1, Wout1, Win2, Wout2):
    b_blk, d = x.shape

    def body(x_ref, win0_ref, wout0_ref, win1_ref, wout1_ref, win2_ref,
             wout2_ref, out_ref, xbuf, psend, rsbuf,
             ag_send, ag_recv, rs_send, rs_recv):
        me = lax.axis_index("i")

        bar = pltpu.get_barrier_semaphore()
        for off in range(1, N_DEV):
            t = (me + off) % N_DEV
            pl.semaphore_signal(bar, inc=1, device_id=(t,),
                                device_id_type=pl.DeviceIdType.MESH)
        pl.semaphore_wait(bar, N_DEV - 1)

        xbuf[me] = x_ref[:, :].astype(jnp.bfloat16)

        wins = [win0_ref, win1_ref, win2_ref]
        wouts = [wout0_ref, wout1_ref, wout2_ref]

        def ag_push(target):
            return pltpu.make_async_remote_copy(
                src_ref=xbuf.at[me], dst_ref=xbuf.at[me],
                send_sem=ag_send.at[target], recv_sem=ag_recv.at[me],
                device_id=(target,), device_id_type=pl.DeviceIdType.MESH)

        def ag_recv_desc(sender):
            return pltpu.make_async_remote_copy(
                src_ref=xbuf.at[sender], dst_ref=xbuf.at[sender],
                send_sem=ag_send.at[sender], recv_sem=ag_recv.at[sender],
                device_id=(sender,), device_id_type=pl.DeviceIdType.MESH)

        acc = None
        for l in range(3):
            ag_rdmas = []
            for off in range(1, N_DEV):
                r = ag_push((me + off) % N_DEV)
                r.start()
                ag_rdmas.append(r)

            w_in = wins[l][:, :].astype(jnp.bfloat16)
            w_out = wouts[l][:, :].astype(jnp.bfloat16)

            def chunk_partial(offs):
                xc = jnp.concatenate(
                    [xbuf[(me + o) % N_DEV] for o in offs], axis=0)
                h = jnp.dot(xc, w_in, preferred_element_type=jnp.float32)
                h = jnp.maximum(h, 0.0).astype(jnp.bfloat16)
                return jnp.dot(h, w_out, preferred_element_type=jnp.float32)

            rs_rdmas = []
            for off in (1, 2, 3):
                ag_recv_desc((me + off) % N_DEV).wait_recv()
            p1 = chunk_partial((0, 1, 2, 3))
            for i, off in enumerate((0, 1, 2, 3)):
                s = (me + off) % N_DEV
                psend[s] = p1[i * b_blk:(i + 1) * b_blk].astype(jnp.bfloat16)
                if off == 0:
                    continue
                r = pltpu.make_async_remote_copy(
                    src_ref=psend.at[s], dst_ref=rsbuf.at[me],
                    send_sem=rs_send.at[s], recv_sem=rs_recv.at[me],
                    device_id=(s,), device_id_type=pl.DeviceIdType.MESH)
                r.start()
                rs_rdmas.append(r)

            for off in (4, 5, 6, 7):
                ag_recv_desc((me + off) % N_DEV).wait_recv()
            p2 = chunk_partial((4, 5, 6, 7))
            for i, off in enumerate((4, 5, 6, 7)):
                s = (me + off) % N_DEV
                psend[s] = p2[i * b_blk:(i + 1) * b_blk].astype(jnp.bfloat16)
                r = pltpu.make_async_remote_copy(
                    src_ref=psend.at[s], dst_ref=rsbuf.at[me],
                    send_sem=rs_send.at[s], recv_sem=rs_recv.at[me],
                    device_id=(s,), device_id_type=pl.DeviceIdType.MESH)
                r.start()
                rs_rdmas.append(r)

            acc = p1[0:b_blk]
            for off in range(1, N_DEV):
                s = (me + off) % N_DEV
                recv = pltpu.make_async_remote_copy(
                    src_ref=rsbuf.at[s], dst_ref=rsbuf.at[s],
                    send_sem=rs_send.at[s], recv_sem=rs_recv.at[s],
                    device_id=(s,), device_id_type=pl.DeviceIdType.MESH)
                recv.wait_recv()
                acc = acc + rsbuf[s].astype(jnp.float32)

            for r in ag_rdmas + rs_rdmas:
                r.wait_send()

            xbuf[me] = acc.astype(jnp.bfloat16)

        fin_rdmas = []
        for off in range(1, N_DEV):
            r = ag_push((me + off) % N_DEV)
            r.start()
            fin_rdmas.append(r)
        out_ref[pl.ds(me * b_blk, b_blk), :] = acc
        for off in range(1, N_DEV):
            s = (me + off) % N_DEV
            ag_recv_desc(s).wait_recv()
            out_ref[pl.ds(s * b_blk, b_blk), :] = xbuf[s].astype(jnp.float32)
        for r in fin_rdmas:
            r.wait_send()

    return pl.pallas_call(
        body,
        out_shape=jax.ShapeDtypeStruct((N_DEV * b_blk, d), jnp.float32),
        in_specs=[pl.BlockSpec(memory_space=pltpu.VMEM)] * 7,
        out_specs=pl.BlockSpec(memory_space=pltpu.VMEM),
        scratch_shapes=[
            pltpu.VMEM((N_DEV, b_blk, d), jnp.bfloat16),
            pltpu.VMEM((N_DEV, b_blk, d), jnp.bfloat16),
            pltpu.VMEM((N_DEV, b_blk, d), jnp.bfloat16),
            pltpu.SemaphoreType.DMA((N_DEV,)),
            pltpu.SemaphoreType.DMA((N_DEV,)),
            pltpu.SemaphoreType.DMA((N_DEV,)),
            pltpu.SemaphoreType.DMA((N_DEV,)),
        ],
        compiler_params=pltpu.CompilerParams(collective_id=0),
    )(x, Win0, Wout0, Win1, Wout1, Win2, Wout2)


# device time: 51592 ns/iter; 1.1020x vs baseline; 1.0717x over previous
import jax
import jax.numpy as jnp
from jax import lax
from jax.experimental import pallas as pl
from jax.experimental.pallas import tpu as pltpu

N_DEV = 8


def kernel(x, Win0, Wout0, Win1, Wout1, Win2, Wout2):
    b_blk, d = x.shape

    def body(x_ref, win0_ref, wout0_ref, win1_ref, wout1_ref, win2_ref,
             wout2_ref, out_ref, xbuf, psend, rsbuf,
             ag_send, ag_recv, rs_send, rs_recv):
        me = lax.axis_index("i")

        bar = pltpu.get_barrier_semaphore()
        for off in range(1, N_DEV):
            t = (me + off) % N_DEV
            pl.semaphore_signal(bar, inc=1, device_id=(t,),
                                device_id_type=pl.DeviceIdType.MESH)
        pl.semaphore_wait(bar, N_DEV - 1)

        xbuf[me] = x_ref[:, :].astype(jnp.bfloat16)

        wins = [win0_ref, win1_ref, win2_ref]
        wouts = [wout0_ref, wout1_ref, wout2_ref]

        def ag_push(target):
            return pltpu.make_async_remote_copy(
                src_ref=xbuf.at[me], dst_ref=xbuf.at[me],
                send_sem=ag_send.at[target], recv_sem=ag_recv.at[me],
                device_id=(target,), device_id_type=pl.DeviceIdType.MESH)

        def ag_recv_desc(sender):
            return pltpu.make_async_remote_copy(
                src_ref=xbuf.at[sender], dst_ref=xbuf.at[sender],
                send_sem=ag_send.at[sender], recv_sem=ag_recv.at[sender],
                device_id=(sender,), device_id_type=pl.DeviceIdType.MESH)

        acc = None
        for l in range(3):
            ag_rdmas = []
            for off in range(1, N_DEV):
                r = ag_push((me + off) % N_DEV)
                r.start()
                ag_rdmas.append(r)

            w_in = wins[l][:, :].astype(jnp.bfloat16)
            w_out = wouts[l][:, :].astype(jnp.bfloat16)

            def chunk_partial(offs):
                xc = jnp.concatenate(
                    [xbuf[(me + o) % N_DEV] for o in offs], axis=0)
                h = jnp.dot(xc, w_in, preferred_element_type=jnp.float32)
                h = jnp.maximum(h, 0.0).astype(jnp.bfloat16)
                return jnp.dot(h, w_out, preferred_element_type=jnp.float32)

            rs_rdmas = []
            for off in (7, 6, 5):
                ag_recv_desc((me + off) % N_DEV).wait_recv()
            p1 = chunk_partial((0, 7, 6, 5))
            for i, off in enumerate((0, 7, 6, 5)):
                s = (me + off) % N_DEV
                psend[s] = p1[i * b_blk:(i + 1) * b_blk].astype(jnp.bfloat16)
                if off == 0:
                    continue
                r = pltpu.make_async_remote_copy(
                    src_ref=psend.at[s], dst_ref=rsbuf.at[me],
                    send_sem=rs_send.at[s], recv_sem=rs_recv.at[me],
                    device_id=(s,), device_id_type=pl.DeviceIdType.MESH)
                r.start()
                rs_rdmas.append(r)

            for off in (4, 3, 2, 1):
                ag_recv_desc((me + off) % N_DEV).wait_recv()
            p2 = chunk_partial((4, 3, 2, 1))
            for i, off in enumerate((4, 3, 2, 1)):
                s = (me + off) % N_DEV
                psend[s] = p2[i * b_blk:(i + 1) * b_blk].astype(jnp.bfloat16)
                r = pltpu.make_async_remote_copy(
                    src_ref=psend.at[s], dst_ref=rsbuf.at[me],
                    send_sem=rs_send.at[s], recv_sem=rs_recv.at[me],
                    device_id=(s,), device_id_type=pl.DeviceIdType.MESH)
                r.start()
                rs_rdmas.append(r)

            acc = p1[0:b_blk]
            for off in range(1, N_DEV):
                s = (me + off) % N_DEV
                recv = pltpu.make_async_remote_copy(
                    src_ref=rsbuf.at[s], dst_ref=rsbuf.at[s],
                    send_sem=rs_send.at[s], recv_sem=rs_recv.at[s],
                    device_id=(s,), device_id_type=pl.DeviceIdType.MESH)
                recv.wait_recv()
                acc = acc + rsbuf[s].astype(jnp.float32)

            for r in ag_rdmas + rs_rdmas:
                r.wait_send()

            xbuf[me] = acc.astype(jnp.bfloat16)

        fin_rdmas = []
        for off in range(1, N_DEV):
            r = ag_push((me + off) % N_DEV)
            r.start()
            fin_rdmas.append(r)
        out_ref[pl.ds(me * b_blk, b_blk), :] = acc
        for off in range(N_DEV - 1, 0, -1):
            s = (me + off) % N_DEV
            ag_recv_desc(s).wait_recv()
            out_ref[pl.ds(s * b_blk, b_blk), :] = xbuf[s].astype(jnp.float32)
        for r in fin_rdmas:
            r.wait_send()

    return pl.pallas_call(
        body,
        out_shape=jax.ShapeDtypeStruct((N_DEV * b_blk, d), jnp.float32),
        in_specs=[pl.BlockSpec(memory_space=pltpu.VMEM)] * 7,
        out_specs=pl.BlockSpec(memory_space=pltpu.VMEM),
        scratch_shapes=[
            pltpu.VMEM((N_DEV, b_blk, d), jnp.bfloat16),
            pltpu.VMEM((N_DEV, b_blk, d), jnp.bfloat16),
            pltpu.VMEM((N_DEV, b_blk, d), jnp.bfloat16),
            pltpu.SemaphoreType.DMA((N_DEV,)),
            pltpu.SemaphoreType.DMA((N_DEV,)),
            pltpu.SemaphoreType.DMA((N_DEV,)),
            pltpu.SemaphoreType.DMA((N_DEV,)),
        ],
        compiler_params=pltpu.CompilerParams(collective_id=0),
    )(x, Win0, Wout0, Win1, Wout1, Win2, Wout2)
